# baseline (speedup 1.0000x reference)
import jax
import jax.numpy as jnp
import numpy as np
from jax import lax
from jax.experimental import pallas as pl
from jax.experimental.pallas import tpu as pltpu
from jax.experimental.pallas import tpu_sc as plsc

N = 10000
E = 320000
D = 128
NC = 2
NS = 16
DH = D // NC
EPS = E // NS
C = 80
K = EPS // C
NP = 10240
RPS = NP // NS
ZR = 128
L = 16

_PERM = np.array(
    list(range(0, 32, 2)) + list(range(1, 32, 2))
    + list(range(32, 64, 2)) + list(range(33, 64, 2)), dtype=np.int32)


def _sc_aggregate_body(x_hbm, src_hbm, dst_hbm, adj_hbm, out_hbm,
                       src_v, dst_v, adj_v,
                       braw0, braw1, braw2, braw3,
                       rf0, rf1, rf2, rf3,
                       acc, sem0, sem1, sem2, sem3,
                       ssem0, ssem1, ssem2, ssem3):
    c = lax.axis_index("core")
    s = lax.axis_index("subcore")
    braws = (braw0, braw1, braw2, braw3)
    rfs = (rf0, rf1, rf2, rf3)
    sems = (sem0, sem1, sem2, sem3)
    ssems = (ssem0, ssem1, ssem2, ssem3)

    @pl.loop(0, C)
    def _(i):
        for b in range(4):
            for q in range(DH // L):
                rfs[b][i, pl.ds(q * L, L)] = jnp.zeros((L,), jnp.float32)

    for b in range(RPS // C):
        pltpu.sync_copy(rfs[b % 4], acc.at[pl.ds(s * RPS + b * C, C)])
    plsc.subcore_barrier()

    pltpu.sync_copy(src_hbm.at[s], src_v)
    pltpu.sync_copy(dst_hbm.at[s], dst_v)
    pltpu.sync_copy(adj_hbm.at[s], adj_v)

    @pl.loop(0, K)
    def _(k):
        for g in range(C // L):
            sl = pl.ds(g * L, L)
            src_v[k, sl] = src_v[k, sl] * 2 + c

    def scale(u, i):
        def one_edge(row, a):
            for h in range(2):
                w = plsc.bitcast(braws[u][row, pl.ds(h * 2 * L, 2 * L)],
                                 jnp.int32)
                even = plsc.bitcast(w << 16, jnp.float32)
                odd = plsc.bitcast(w & jnp.int32(-65536), jnp.float32)
                rfs[u][row, pl.ds((2 * h) * L, L)] = even * a
                rfs[u][row, pl.ds((2 * h + 1) * L, L)] = odd * a

        def edge_pair(base, wj):
            a0 = lax.bitcast_convert_type(wj << 16, jnp.float32)
            a1 = lax.bitcast_convert_type(wj & jnp.int32(-65536),
                                          jnp.float32)
            one_edge(base, a0)
            one_edge(base + 1, a1)

        for gg in range(2):
            avw = plsc.bitcast(adj_v[i, pl.ds(gg * 2 * L, 2 * L)],
                               jnp.int32)
            for j in range(L):
                edge_pair(gg * 2 * L + 2 * j, avw[j])
        avw = plsc.bitcast(adj_v[i, pl.ds(C - 2 * L, 2 * L)], jnp.int32)
        for j in range(L // 2, L):
            edge_pair(C - 2 * L + 2 * j, avw[j])

    def process(i, u):
        pltpu.make_async_copy(x_hbm.at[src_v.at[i]], braws[u],
                              sems[u]).wait()
        scale(u, i)
        pltpu.async_copy(rfs[u], acc.at[dst_v.at[i]], ssems[u], add=True)

    def drain_scatter(i, u):
        pltpu.make_async_copy(rfs[u], acc.at[dst_v.at[i]],
                              ssems[u]).wait()

    pltpu.async_copy(x_hbm.at[src_v.at[0]], braws[0], sems[0])
    pltpu.async_copy(x_hbm.at[src_v.at[1]], braws[1], sems[1])

    @pl.loop(0, K - 2, step=4)
    def _(k):
        for u in range(4):
            i = k + u
            jn = (u + 2) % 4

            @pl.when(i >= 2)
            def _():
                drain_scatter(i - 2, jn)

            pltpu.async_copy(x_hbm.at[src_v.at[i + 2]], braws[jn], sems[jn])
            process(i, u)

    process(K - 2, (K - 2) % 4)
    process(K - 1, (K - 1) % 4)
    drain_scatter(K - 4, (K - 4) % 4)
    drain_scatter(K - 3, (K - 3) % 4)
    drain_scatter(K - 2, (K - 2) % 4)
    drain_scatter(K - 1, (K - 1) % 4)

    plsc.subcore_barrier()
    pltpu.sync_copy(acc.at[pl.ds(s * RPS, RPS)],
                    out_hbm.at[c, pl.ds(s * RPS, RPS)])


def _sc_aggregate(x2, srcs, dst, adj):
    mesh = plsc.VectorSubcoreMesh(core_axis_name="core",
                                  subcore_axis_name="subcore")
    return pl.kernel(
        _sc_aggregate_body,
        out_type=jax.ShapeDtypeStruct((NC, NP, DH), jnp.float32),
        mesh=mesh,
        compiler_params=pltpu.CompilerParams(use_tc_tiling_on_sc=False, needs_layout_passes=False),
        scratch_types=[
            pltpu.VMEM((K, C), jnp.int32),
            pltpu.VMEM((K, C), jnp.int32),
            pltpu.VMEM((K, C), jnp.bfloat16),
            pltpu.VMEM((C, DH), jnp.bfloat16),
            pltpu.VMEM((C, DH), jnp.bfloat16),
            pltpu.VMEM((C, DH), jnp.bfloat16),
            pltpu.VMEM((C, DH), jnp.bfloat16),
            pltpu.VMEM((C, DH), jnp.float32),
            pltpu.VMEM((C, DH), jnp.float32),
            pltpu.VMEM((C, DH), jnp.float32),
            pltpu.VMEM((C, DH), jnp.float32),
            pltpu.VMEM_SHARED((NP, DH), jnp.float32),
            pltpu.SemaphoreType.DMA,
            pltpu.SemaphoreType.DMA,
            pltpu.SemaphoreType.DMA,
            pltpu.SemaphoreType.DMA,
            pltpu.SemaphoreType.DMA,
            pltpu.SemaphoreType.DMA,
            pltpu.SemaphoreType.DMA,
            pltpu.SemaphoreType.DMA,
        ],
    )(x2, srcs, dst, adj)


def _tc_matmul_body(p_ref, w_ref, o_ref):
    acc = (jnp.dot(p_ref[0], w_ref[0], preferred_element_type=jnp.float32)
           + jnp.dot(p_ref[1], w_ref[1], preferred_element_type=jnp.float32))
    o_ref[...] = jnp.maximum(acc, 0.0)


def _tc_matmul(partials, W0):
    R = 1000
    return pl.pallas_call(
        _tc_matmul_body,
        grid=(N // R,),
        in_specs=[
            pl.BlockSpec((NC, R, DH), lambda i: (0, i, 0)),
            pl.BlockSpec((NC, DH, D), lambda i: (0, 0, 0)),
        ],
        out_specs=pl.BlockSpec((R, D), lambda i: (i, 0)),
        out_shape=jax.ShapeDtypeStruct((N, D), jnp.float32),
    )(partials, W0)


def kernel(x, edge_index, adj_values, W0):
    src = edge_index[0].astype(jnp.int32)
    dst = edge_index[1].astype(jnp.int32)
    x2 = x.astype(jnp.bfloat16).reshape(NC * N, DH)
    srcs = src.reshape(NS, K, C)
    dstr = dst.reshape(NS, K, C)
    adjr = adj_values.astype(jnp.bfloat16).reshape(NS, K, C)
    w2 = W0.reshape(NC, DH, D)[:, _PERM, :]
    partials = _sc_aggregate(x2, srcs, dstr, adjr)
    return _tc_matmul(partials, w2)


# speedup vs baseline: 1.0270x; 1.0270x over previous
import jax
import jax.numpy as jnp
from jax import lax
from jax.experimental import pallas as pl
from jax.experimental.pallas import tpu as pltpu
from jax.experimental.pallas import tpu_sc as plsc

N = 10000
E = 320000
D = 128
NC = 2
NS = 16
DH = D // NC
EPS = E // NS
C = 80
K = EPS // C
NP = 10240
RPS = NP // NS
ZR = 128
L = 16


def _sc_aggregate_body(x_hbm, src_hbm, dst_hbm, adj_hbm, out_hbm,
                       src_v, dst_v, adj_v, rows0, rows1, rows2, rows3,
                       zbuf, acc, sem0, sem1, sem2, sem3,
                       ssem0, ssem1, ssem2, ssem3):
    c = lax.axis_index("core")
    s = lax.axis_index("subcore")
    rows = (rows0, rows1, rows2, rows3)
    sems = (sem0, sem1, sem2, sem3)
    ssems = (ssem0, ssem1, ssem2, ssem3)

    @pl.loop(0, ZR)
    def _(i):
        for q in range(DH // L):
            zbuf[i, pl.ds(q * L, L)] = jnp.zeros((L,), jnp.float32)

    for k in range(RPS // ZR):
        pltpu.sync_copy(zbuf, acc.at[pl.ds(s * RPS + k * ZR, ZR)])
    plsc.subcore_barrier()

    pltpu.sync_copy(src_hbm.at[s], src_v)
    pltpu.sync_copy(dst_hbm.at[s], dst_v)
    pltpu.sync_copy(adj_hbm.at[s], adj_v)

    @pl.loop(0, K)
    def _(k):
        for g in range(C // L):
            sl = pl.ds(g * L, L)
            src_v[k, sl] = src_v[k, sl] * 2 + c

    def scale(buf, i):
        for g in range(C // L):
            av = adj_v[i, pl.ds(g * L, L)]
            for j in range(L):
                a = av[j]
                row = g * L + j
                for q in range(DH // L):
                    sl = pl.ds(q * L, L)
                    buf[row, sl] = buf[row, sl] * a

    def process(i, u):
        pltpu.make_async_copy(x_hbm.at[src_v.at[i]], rows[u], sems[u]).wait()
        scale(rows[u], i)
        pltpu.async_copy(rows[u], acc.at[dst_v.at[i]], ssems[u], add=True)

    def drain_scatter(u):
        pltpu.make_async_copy(rows[u], acc.at[dst_v.at[0]],
                              ssems[u]).wait()

    for u in (2, 3):
        pltpu.async_copy(zbuf.at[pl.ds(0, C)], acc.at[dst_v.at[u]],
                         ssems[u], add=True)
    pltpu.async_copy(x_hbm.at[src_v.at[0]], rows[0], sems[0])
    pltpu.async_copy(x_hbm.at[src_v.at[1]], rows[1], sems[1])

    @pl.loop(0, K - 2, step=4)
    def _(k):
        for u in range(4):
            i = k + u
            jn = (u + 2) % 4
            drain_scatter(jn)
            pltpu.async_copy(x_hbm.at[src_v.at[i + 2]], rows[jn], sems[jn])
            process(i, u)

    process(K - 2, (K - 2) % 4)
    process(K - 1, (K - 1) % 4)
    drain_scatter((K - 4) % 4)
    drain_scatter((K - 3) % 4)
    drain_scatter((K - 2) % 4)
    drain_scatter((K - 1) % 4)

    plsc.subcore_barrier()
    pltpu.sync_copy(acc.at[pl.ds(s * RPS, RPS)],
                    out_hbm.at[c, pl.ds(s * RPS, RPS)])


def _sc_aggregate(x2, srcs, dst, adj):
    mesh = plsc.VectorSubcoreMesh(core_axis_name="core",
                                  subcore_axis_name="subcore")
    return pl.kernel(
        _sc_aggregate_body,
        out_type=jax.ShapeDtypeStruct((NC, NP, DH), jnp.float32),
        mesh=mesh,
        compiler_params=pltpu.CompilerParams(use_tc_tiling_on_sc=False),
        scratch_types=[
            pltpu.VMEM((K, C), jnp.int32),
            pltpu.VMEM((K, C), jnp.int32),
            pltpu.VMEM((K, C), jnp.float32),
            pltpu.VMEM((C, DH), jnp.float32),
            pltpu.VMEM((C, DH), jnp.float32),
            pltpu.VMEM((C, DH), jnp.float32),
            pltpu.VMEM((C, DH), jnp.float32),
            pltpu.VMEM((ZR, DH), jnp.float32),
            pltpu.VMEM_SHARED((NP, DH), jnp.float32),
            pltpu.SemaphoreType.DMA,
            pltpu.SemaphoreType.DMA,
            pltpu.SemaphoreType.DMA,
            pltpu.SemaphoreType.DMA,
            pltpu.SemaphoreType.DMA,
            pltpu.SemaphoreType.DMA,
            pltpu.SemaphoreType.DMA,
            pltpu.SemaphoreType.DMA,
        ],
    )(x2, srcs, dst, adj)


def _tc_matmul_body(p_ref, w_ref, o_ref):
    acc = (jnp.dot(p_ref[0], w_ref[0], preferred_element_type=jnp.float32)
           + jnp.dot(p_ref[1], w_ref[1], preferred_element_type=jnp.float32))
    o_ref[...] = jnp.maximum(acc, 0.0)


def _tc_matmul(partials, W0):
    R = 1000
    return pl.pallas_call(
        _tc_matmul_body,
        grid=(N // R,),
        in_specs=[
            pl.BlockSpec((NC, R, DH), lambda i: (0, i, 0)),
            pl.BlockSpec((NC, DH, D), lambda i: (0, 0, 0)),
        ],
        out_specs=pl.BlockSpec((R, D), lambda i: (i, 0)),
        out_shape=jax.ShapeDtypeStruct((N, D), jnp.float32),
    )(partials, W0)


def kernel(x, edge_index, adj_values, W0):
    src = edge_index[0].astype(jnp.int32)
    dst = edge_index[1].astype(jnp.int32)
    x2 = x.reshape(NC * N, DH)
    srcs = src.reshape(NS, K, C)
    dstr = dst.reshape(NS, K, C)
    adjr = adj_values.reshape(NS, K, C)
    w2 = W0.reshape(NC, DH, D)
    partials = _sc_aggregate(x2, srcs, dstr, adjr)
    return _tc_matmul(partials, w2)


# speedup vs baseline: 1.0443x; 1.0168x over previous
import jax
import jax.numpy as jnp
from jax import lax
from jax.experimental import pallas as pl
from jax.experimental.pallas import tpu as pltpu
from jax.experimental.pallas import tpu_sc as plsc

N = 10000
E = 320000
D = 128
NC = 2
NS = 16
DH = D // NC
EPS = E // NS
C = 80
K = EPS // C
NP = 10240
RPS = NP // NS
ZR = 128
L = 16


def _sc_aggregate_body(x_hbm, src_hbm, dst_hbm, adj_hbm, out_hbm,
                       src_v, dst_v, adj_v, rows0, rows1, rows2, rows3,
                       zbuf, acc, sem0, sem1, sem2, sem3,
                       ssem0, ssem1, ssem2, ssem3):
    c = lax.axis_index("core")
    s = lax.axis_index("subcore")
    rows = (rows0, rows1, rows2, rows3)
    sems = (sem0, sem1, sem2, sem3)
    ssems = (ssem0, ssem1, ssem2, ssem3)

    @pl.loop(0, ZR)
    def _(i):
        for q in range(DH // L):
            zbuf[i, pl.ds(q * L, L)] = jnp.zeros((L,), jnp.float32)

    for k in range(RPS // ZR):
        pltpu.sync_copy(zbuf, acc.at[pl.ds(s * RPS + k * ZR, ZR)])
    plsc.subcore_barrier()

    pltpu.sync_copy(src_hbm.at[s], src_v)
    pltpu.sync_copy(dst_hbm.at[s], dst_v)
    pltpu.sync_copy(adj_hbm.at[s], adj_v)

    @pl.loop(0, K)
    def _(k):
        for g in range(C // L):
            sl = pl.ds(g * L, L)
            src_v[k, sl] = src_v[k, sl] * 2 + c

    def scale(buf, i):
        for g in range(C // L):
            av = adj_v[i, pl.ds(g * L, L)]
            for j in range(L):
                a = av[j]
                row = g * L + j
                for q in range(DH // L):
                    sl = pl.ds(q * L, L)
                    buf[row, sl] = buf[row, sl] * a

    def process(i, u):
        pltpu.make_async_copy(x_hbm.at[src_v.at[i]], rows[u], sems[u]).wait()
        scale(rows[u], i)
        pltpu.async_copy(rows[u], acc.at[dst_v.at[i]], ssems[u], add=True)

    def drain_scatter(i, u):
        pltpu.make_async_copy(rows[u], acc.at[dst_v.at[i]],
                              ssems[u]).wait()

    pltpu.async_copy(x_hbm.at[src_v.at[0]], rows[0], sems[0])
    pltpu.async_copy(x_hbm.at[src_v.at[1]], rows[1], sems[1])

    @pl.loop(0, K - 2, step=4)
    def _(k):
        for u in range(4):
            i = k + u
            jn = (u + 2) % 4

            @pl.when(i >= 2)
            def _():
                drain_scatter(i - 2, jn)

            pltpu.async_copy(x_hbm.at[src_v.at[i + 2]], rows[jn], sems[jn])
            process(i, u)

    process(K - 2, (K - 2) % 4)
    process(K - 1, (K - 1) % 4)
    drain_scatter(K - 4, (K - 4) % 4)
    drain_scatter(K - 3, (K - 3) % 4)
    drain_scatter(K - 2, (K - 2) % 4)
    drain_scatter(K - 1, (K - 1) % 4)

    plsc.subcore_barrier()
    pltpu.sync_copy(acc.at[pl.ds(s * RPS, RPS)],
                    out_hbm.at[c, pl.ds(s * RPS, RPS)])


def _sc_aggregate(x2, srcs, dst, adj):
    mesh = plsc.VectorSubcoreMesh(core_axis_name="core",
                                  subcore_axis_name="subcore")
    return pl.kernel(
        _sc_aggregate_body,
        out_type=jax.ShapeDtypeStruct((NC, NP, DH), jnp.float32),
        mesh=mesh,
        compiler_params=pltpu.CompilerParams(use_tc_tiling_on_sc=False),
        scratch_types=[
            pltpu.VMEM((K, C), jnp.int32),
            pltpu.VMEM((K, C), jnp.int32),
            pltpu.VMEM((K, C), jnp.float32),
            pltpu.VMEM((C, DH), jnp.float32),
            pltpu.VMEM((C, DH), jnp.float32),
            pltpu.VMEM((C, DH), jnp.float32),
            pltpu.VMEM((C, DH), jnp.float32),
            pltpu.VMEM((ZR, DH), jnp.float32),
            pltpu.VMEM_SHARED((NP, DH), jnp.float32),
            pltpu.SemaphoreType.DMA,
            pltpu.SemaphoreType.DMA,
            pltpu.SemaphoreType.DMA,
            pltpu.SemaphoreType.DMA,
            pltpu.SemaphoreType.DMA,
            pltpu.SemaphoreType.DMA,
            pltpu.SemaphoreType.DMA,
            pltpu.SemaphoreType.DMA,
        ],
    )(x2, srcs, dst, adj)


def _tc_matmul_body(p_ref, w_ref, o_ref):
    acc = (jnp.dot(p_ref[0], w_ref[0], preferred_element_type=jnp.float32)
           + jnp.dot(p_ref[1], w_ref[1], preferred_element_type=jnp.float32))
    o_ref[...] = jnp.maximum(acc, 0.0)


def _tc_matmul(partials, W0):
    R = 1000
    return pl.pallas_call(
        _tc_matmul_body,
        grid=(N // R,),
        in_specs=[
            pl.BlockSpec((NC, R, DH), lambda i: (0, i, 0)),
            pl.BlockSpec((NC, DH, D), lambda i: (0, 0, 0)),
        ],
        out_specs=pl.BlockSpec((R, D), lambda i: (i, 0)),
        out_shape=jax.ShapeDtypeStruct((N, D), jnp.float32),
    )(partials, W0)


def kernel(x, edge_index, adj_values, W0):
    src = edge_index[0].astype(jnp.int32)
    dst = edge_index[1].astype(jnp.int32)
    x2 = x.reshape(NC * N, DH)
    srcs = src.reshape(NS, K, C)
    dstr = dst.reshape(NS, K, C)
    adjr = adj_values.reshape(NS, K, C)
    w2 = W0.reshape(NC, DH, D)
    partials = _sc_aggregate(x2, srcs, dstr, adjr)
    return _tc_matmul(partials, w2)
